# baseline (speedup 1.0000x reference)
import functools

import jax
import jax.numpy as jnp
from jax import lax
from jax.experimental import pallas as pl
from jax.experimental.pallas import tpu as pltpu
from jax.experimental.pallas import tpu_sc as plsc

N_IN = 262144
N_OUT = 262144
K = 5
F_IN = 4
F_OUT = 32
KC = K * F_IN

N_TPAD = N_IN + 1024
TOTAL = N_OUT * KC
NC, NS = 2, 16
NW = NC * NS
TOT_W = TOTAL // NW
CHUNK = 8192
GSUB = 128
N_CHUNKS = TOT_W // CHUNK


def _table_body(dr_ref, t_ref, s_ref, out_ref):
    t = t_ref[...]
    s = s_ref[...]
    arg = jnp.zeros_like(t)
    for c in range(F_IN):
        arg += jnp.where(s == c, dr_ref[c], 0.0)
    w = jnp.exp(arg * t)
    bits = lax.bitcast_convert_type(w, jnp.int32)
    packed = (bits & ~3) | (s & 3)
    out_ref[...] = jnp.where(s >= 0, packed, 0)


def _build_table(dr, t_pad, s_pad):
    rows = N_TPAD // 128
    return pl.pallas_call(
        _table_body,
        out_shape=jax.ShapeDtypeStruct((rows, 128), jnp.int32),
        in_specs=[
            pl.BlockSpec(memory_space=pltpu.MemorySpace.SMEM),
            pl.BlockSpec((rows, 128), lambda: (0, 0)),
            pl.BlockSpec((rows, 128), lambda: (0, 0)),
        ],
        out_specs=pl.BlockSpec((rows, 128), lambda: (0, 0)),
    )(dr, t_pad.reshape(rows, 128), s_pad.reshape(rows, 128))


def _gather_body(table_hbm, ids_hbm, out_hbm, idx_v, rows_v, sem):
    wid = lax.axis_index("s") * NC + lax.axis_index("c")
    base = wid * TOT_W

    def chunk_body(ci, carry):
        off = base + ci * CHUNK
        pltpu.sync_copy(ids_hbm.at[pl.ds(off, CHUNK)], idx_v)

        def fire(m, carry2):
            pltpu.async_copy(
                table_hbm.at[idx_v.at[pl.ds(m * GSUB, GSUB)]],
                rows_v.at[pl.ds(m * GSUB, GSUB)],
                sem,
            )
            return carry2

        lax.fori_loop(0, CHUNK // GSUB, fire, 0)
        pltpu.make_async_copy(table_hbm.at[pl.ds(0, CHUNK)], rows_v, sem).wait()
        pltpu.sync_copy(rows_v, out_hbm.at[pl.ds(off, CHUNK)])
        return carry

    lax.fori_loop(0, N_CHUNKS, chunk_body, 0)


@functools.lru_cache(maxsize=None)
def _make_gather():
    return functools.partial(
        pl.kernel,
        mesh=plsc.VectorSubcoreMesh(
            core_axis_name="c", subcore_axis_name="s",
            num_cores=NC, num_subcores=NS,
        ),
        out_type=jax.ShapeDtypeStruct((TOTAL,), jnp.int32),
        scratch_types=[
            pltpu.VMEM((CHUNK,), jnp.int32),
            pltpu.VMEM((CHUNK,), jnp.int32),
            pltpu.SemaphoreType.DMA,
        ],
    )(_gather_body)


def _sc_gather(table, ids_flat):
    return _make_gather()(table, ids_flat)


_JB = 8
_NB = N_OUT // 128


def _out_body(dr_ref, g_ref, t_ref, a_ref, b_ref, o_ref):
    bits = g_ref[...]
    w = lax.bitcast_convert_type(bits & ~3, jnp.float32)
    ch = bits & 3
    cpat = lax.broadcasted_iota(jnp.int32, (KC, _JB, 128), 0) % F_IN
    t = t_ref[...]
    e4 = jnp.stack([jnp.exp(-dr_ref[c] * t) for c in range(F_IN)])
    e20 = jnp.concatenate([e4] * K, axis=0)
    vals = jnp.where(ch == cpat, w * e20, 0.0)
    v160 = vals.reshape(KC * _JB, 128)
    res = lax.dot_general(a_ref[...], v160, (((1,), (0,)), ((), ())),
                          preferred_element_type=jnp.float32)
    res = res + b_ref[...]
    res4 = res.reshape(_JB, F_OUT // 8, 8, 128)
    o_ref[...] = jnp.transpose(res4, (1, 0, 2, 3))


def _finish(dr, g3, tout2, a_mat, bias_col):
    grid = (_NB // _JB,)
    return pl.pallas_call(
        _out_body,
        grid=grid,
        out_shape=jax.ShapeDtypeStruct((F_OUT // 8, _NB, 8, 128), jnp.float32),
        in_specs=[
            pl.BlockSpec(memory_space=pltpu.MemorySpace.SMEM),
            pl.BlockSpec((KC, _JB, 128), lambda i: (0, i, 0)),
            pl.BlockSpec((_JB, 128), lambda i: (i, 0)),
            pl.BlockSpec((_JB * F_OUT, KC * _JB), lambda i: (0, 0)),
            pl.BlockSpec((_JB * F_OUT, 1), lambda i: (0, 0)),
        ],
        out_specs=pl.BlockSpec((F_OUT // 8, _JB, 8, 128), lambda i: (0, i, 0, 0)),
    )(dr, g3, tout2, a_mat, bias_col)


def kernel(times_in, times_out, segment_filter_ids, one_hot_predecessor_ids,
           decay_rate, kernel, bias):
    dr = jax.nn.softplus(decay_rate)

    pad = N_TPAD - N_IN
    t_pad = jnp.pad(times_in, (0, pad))
    s_pad = jnp.pad(segment_filter_ids, (0, pad), constant_values=-1)

    table = _build_table(dr, t_pad, s_pad).reshape(-1)
    ids_flat = jnp.transpose(one_hot_predecessor_ids, (1, 2, 0)).reshape(-1)
    g = _sc_gather(table, ids_flat)

    m2 = kernel.reshape(KC, F_OUT)
    a_mat = jnp.einsum("pf,st->sfpt", m2, jnp.eye(_JB, dtype=m2.dtype))
    a_mat = a_mat.reshape(_JB * F_OUT, KC * _JB)
    bias_col = jnp.tile(bias, (_JB,)).reshape(_JB * F_OUT, 1)

    out4 = _finish(
        dr,
        g.reshape(KC, _NB, 128),
        times_out.reshape(_NB, 128),
        a_mat,
        bias_col,
    )
    return jnp.transpose(out4, (1, 3, 0, 2)).reshape(N_OUT, F_OUT)


# speedup vs baseline: 1.3221x; 1.3221x over previous
import functools

import jax
import jax.numpy as jnp
from jax import lax
from jax.experimental import pallas as pl
from jax.experimental.pallas import tpu as pltpu
from jax.experimental.pallas import tpu_sc as plsc

N_IN = 262144
N_OUT = 262144
K = 5
F_IN = 4
F_OUT = 32
KC = K * F_IN

N_TPAD = N_IN + 1024
TOTAL = N_OUT * KC
NC, NS = 2, 16
NW = NC * NS
TOT_W = TOTAL // NW
CHUNK = 8192
GSUB = 128
N_CHUNKS = TOT_W // CHUNK


def _table_body(dr_ref, t_ref, s_ref, out_ref):
    t = t_ref[...]
    s = s_ref[...]
    arg = jnp.zeros_like(t)
    for c in range(F_IN):
        arg += jnp.where(s == c, dr_ref[c], 0.0)
    w = jnp.exp(arg * t)
    bits = lax.bitcast_convert_type(w, jnp.int32)
    packed = (bits & ~3) | (s & 3)
    out_ref[...] = jnp.where(s >= 0, packed, 0)


def _build_table(dr, t_pad, s_pad):
    rows = N_TPAD // 128
    return pl.pallas_call(
        _table_body,
        out_shape=jax.ShapeDtypeStruct((rows, 128), jnp.int32),
        in_specs=[
            pl.BlockSpec(memory_space=pltpu.MemorySpace.SMEM),
            pl.BlockSpec((rows, 128), lambda: (0, 0)),
            pl.BlockSpec((rows, 128), lambda: (0, 0)),
        ],
        out_specs=pl.BlockSpec((rows, 128), lambda: (0, 0)),
    )(dr, t_pad.reshape(rows, 128), s_pad.reshape(rows, 128))


def _gather_body(table_hbm, ids_hbm, out_hbm, idx_v, rows_v, sem):
    wid = lax.axis_index("s") * NC + lax.axis_index("c")
    base = wid * TOT_W

    def chunk_body(ci, carry):
        off = base + ci * CHUNK
        pltpu.sync_copy(ids_hbm.at[pl.ds(off, CHUNK)], idx_v)

        def fire(m, carry2):
            pltpu.async_copy(
                table_hbm.at[idx_v.at[pl.ds(m * GSUB, GSUB)]],
                rows_v.at[pl.ds(m * GSUB, GSUB)],
                sem,
            )
            return carry2

        lax.fori_loop(0, CHUNK // GSUB, fire, 0)
        pltpu.make_async_copy(table_hbm.at[pl.ds(0, CHUNK)], rows_v, sem).wait()
        pltpu.sync_copy(rows_v, out_hbm.at[pl.ds(off, CHUNK)])
        return carry

    lax.fori_loop(0, N_CHUNKS, chunk_body, 0)


@functools.lru_cache(maxsize=None)
def _make_gather():
    return functools.partial(
        pl.kernel,
        mesh=plsc.VectorSubcoreMesh(
            core_axis_name="c", subcore_axis_name="s",
            num_cores=NC, num_subcores=NS,
        ),
        out_type=jax.ShapeDtypeStruct((TOTAL,), jnp.int32),
        scratch_types=[
            pltpu.VMEM((CHUNK,), jnp.int32),
            pltpu.VMEM((CHUNK,), jnp.int32),
            pltpu.SemaphoreType.DMA,
        ],
    )(_gather_body)


def _sc_gather(table, ids_flat):
    return _make_gather()(table, ids_flat)


_JB = 32
_SG = 8
_NB = N_OUT // 128


def _out_body(dr_ref, g_ref, t_ref, a_ref, b_ref, o_ref):
    bits = g_ref[...]
    w = lax.bitcast_convert_type(bits & ~3, jnp.float32)
    ch = bits & 3
    cpat = lax.broadcasted_iota(jnp.int32, (KC, _JB, 128), 0) % F_IN
    t = t_ref[...]
    e4 = jnp.stack([jnp.exp(-dr_ref[c] * t) for c in range(F_IN)])
    e20 = jnp.concatenate([e4] * K, axis=0)
    vals = jnp.where(ch == cpat, w * e20, 0.0)
    a = a_ref[...]
    b = b_ref[...]
    for q in range(_JB // _SG):
        vq = vals[:, q * _SG:(q + 1) * _SG, :]
        v160 = vq.reshape(KC * _SG, 128)
        res = lax.dot_general(a, v160, (((1,), (0,)), ((), ())),
                              preferred_element_type=jnp.float32)
        res = res + b
        res4 = res.reshape(_SG, F_OUT // 8, 8, 128)
        o_ref[:, q * _SG:(q + 1) * _SG, :, :] = jnp.transpose(res4, (1, 0, 2, 3))


def _finish(dr, g3, tout2, a_mat, bias_col):
    grid = (_NB // _JB,)
    return pl.pallas_call(
        _out_body,
        grid=grid,
        out_shape=jax.ShapeDtypeStruct((F_OUT // 8, _NB, 8, 128), jnp.float32),
        in_specs=[
            pl.BlockSpec(memory_space=pltpu.MemorySpace.SMEM),
            pl.BlockSpec((KC, _JB, 128), lambda i: (0, i, 0)),
            pl.BlockSpec((_JB, 128), lambda i: (i, 0)),
            pl.BlockSpec((_SG * F_OUT, KC * _SG), lambda i: (0, 0)),
            pl.BlockSpec((_SG * F_OUT, 1), lambda i: (0, 0)),
        ],
        out_specs=pl.BlockSpec((F_OUT // 8, _JB, 8, 128), lambda i: (0, i, 0, 0)),
    )(dr, g3, tout2, a_mat, bias_col)


def kernel(times_in, times_out, segment_filter_ids, one_hot_predecessor_ids,
           decay_rate, kernel, bias):
    dr = jax.nn.softplus(decay_rate)

    pad = N_TPAD - N_IN
    t_pad = jnp.pad(times_in, (0, pad))
    s_pad = jnp.pad(segment_filter_ids, (0, pad), constant_values=-1)

    table = _build_table(dr, t_pad, s_pad).reshape(-1)
    ids_flat = jnp.transpose(one_hot_predecessor_ids, (1, 2, 0)).reshape(-1)
    g = _sc_gather(table, ids_flat)

    m2 = kernel.reshape(KC, F_OUT)
    a_mat = jnp.einsum("pf,st->sfpt", m2, jnp.eye(_SG, dtype=m2.dtype))
    a_mat = a_mat.reshape(_SG * F_OUT, KC * _SG)
    bias_col = jnp.tile(bias, (_SG,)).reshape(_SG * F_OUT, 1)

    out4 = _finish(
        dr,
        g.reshape(KC, _NB, 128),
        times_out.reshape(_NB, 128),
        a_mat,
        bias_col,
    )
    return jnp.transpose(out4, (1, 3, 0, 2)).reshape(N_OUT, F_OUT)


# speedup vs baseline: 1.3528x; 1.0233x over previous
import functools

import jax
import jax.numpy as jnp
from jax import lax
from jax.experimental import pallas as pl
from jax.experimental.pallas import tpu as pltpu
from jax.experimental.pallas import tpu_sc as plsc

N_IN = 262144
N_OUT = 262144
K = 5
F_IN = 4
F_OUT = 32
KC = K * F_IN

N_TPAD = N_IN + 1024
TOTAL = N_OUT * KC
NC, NS = 2, 16
NW = NC * NS
TOT_W = TOTAL // NW
CHUNK = 8192
GSUB = 128
N_CHUNKS = TOT_W // CHUNK


def _table_body(dr_ref, t_ref, s_ref, out_ref):
    t = t_ref[...]
    s = s_ref[...]
    arg = jnp.zeros_like(t)
    for c in range(F_IN):
        arg += jnp.where(s == c, dr_ref[c], 0.0)
    w = jnp.exp(arg * t)
    bits = lax.bitcast_convert_type(w, jnp.int32)
    packed = (bits & ~3) | (s & 3)
    out_ref[...] = jnp.where(s >= 0, packed, 0)


def _build_table(dr, t_pad, s_pad):
    rows = N_TPAD // 128
    return pl.pallas_call(
        _table_body,
        out_shape=jax.ShapeDtypeStruct((rows, 128), jnp.int32),
        in_specs=[
            pl.BlockSpec(memory_space=pltpu.MemorySpace.SMEM),
            pl.BlockSpec((rows, 128), lambda: (0, 0)),
            pl.BlockSpec((rows, 128), lambda: (0, 0)),
        ],
        out_specs=pl.BlockSpec((rows, 128), lambda: (0, 0)),
    )(dr, t_pad.reshape(rows, 128), s_pad.reshape(rows, 128))


def _gather_body(table_hbm, ids_hbm, out_hbm, idx_v, rows_v, sem):
    wid = lax.axis_index("s") * NC + lax.axis_index("c")
    base = wid * TOT_W

    def chunk_body(ci, carry):
        off = base + ci * CHUNK
        pltpu.sync_copy(ids_hbm.at[pl.ds(off, CHUNK)], idx_v)

        def fire(m, carry2):
            pltpu.async_copy(
                table_hbm.at[idx_v.at[pl.ds(m * GSUB, GSUB)]],
                rows_v.at[pl.ds(m * GSUB, GSUB)],
                sem,
            )
            return carry2

        lax.fori_loop(0, CHUNK // GSUB, fire, 0)
        pltpu.make_async_copy(table_hbm.at[pl.ds(0, CHUNK)], rows_v, sem).wait()
        pltpu.sync_copy(rows_v, out_hbm.at[pl.ds(off, CHUNK)])
        return carry

    lax.fori_loop(0, N_CHUNKS, chunk_body, 0)


@functools.lru_cache(maxsize=None)
def _make_gather():
    return functools.partial(
        pl.kernel,
        mesh=plsc.VectorSubcoreMesh(
            core_axis_name="c", subcore_axis_name="s",
            num_cores=NC, num_subcores=NS,
        ),
        out_type=jax.ShapeDtypeStruct((TOTAL,), jnp.int32),
        scratch_types=[
            pltpu.VMEM((CHUNK,), jnp.int32),
            pltpu.VMEM((CHUNK,), jnp.int32),
            pltpu.SemaphoreType.DMA,
        ],
    )(_gather_body)


def _sc_gather(table, ids_flat):
    return _make_gather()(table, ids_flat)


_NB = N_OUT // 128
_SB = 128
_JBLK = _SB // F_IN
_QS = 32


def _out_body(dr_ref, g_ref, t_ref, a_ref, b_ref, o_ref):
    bits = g_ref[...]
    w = lax.bitcast_convert_type(bits & ~3, jnp.float32)
    ch = bits & 3
    cpat2 = lax.broadcasted_iota(jnp.int32, (_SB, 128), 0) % F_IN
    t = t_ref[...]
    drsel = jnp.zeros((_SB, 128), jnp.float32)
    for c in range(F_IN):
        drsel += jnp.where(cpat2 == c, dr_ref[c], 0.0)
    e = jnp.exp(-drsel * t)
    vals = jnp.where(ch == cpat2[None], w * e[None], 0.0)
    a = a_ref[...]
    b = b_ref[...]
    for q in range(_SB // _QS):
        vq = vals[:, q * _QS:(q + 1) * _QS, :]
        v160 = vq.reshape(K * _QS, 128)
        res = lax.dot_general(a, v160, (((1,), (0,)), ((), ())),
                              preferred_element_type=jnp.float32)
        res = res + b
        res4 = res.reshape(8, F_OUT // 8, 8, 128)
        o_ref[:, q * 8:(q + 1) * 8, :, :] = jnp.transpose(res4, (1, 0, 2, 3))


def _finish(dr, g3, t_rep, a_mat, bias_col):
    grid = (F_IN * _NB // _SB,)
    return pl.pallas_call(
        _out_body,
        grid=grid,
        out_shape=jax.ShapeDtypeStruct((F_OUT // 8, _NB, 8, 128), jnp.float32),
        in_specs=[
            pl.BlockSpec(memory_space=pltpu.MemorySpace.SMEM),
            pl.BlockSpec((K, _SB, 128), lambda i: (0, i, 0)),
            pl.BlockSpec((_SB, 128), lambda i: (i, 0)),
            pl.BlockSpec((8 * F_OUT, K * _QS), lambda i: (0, 0)),
            pl.BlockSpec((8 * F_OUT, 1), lambda i: (0, 0)),
        ],
        out_specs=pl.BlockSpec((F_OUT // 8, _JBLK, 8, 128), lambda i: (0, i, 0, 0)),
    )(dr, g3, t_rep, a_mat, bias_col)


def kernel(times_in, times_out, segment_filter_ids, one_hot_predecessor_ids,
           decay_rate, kernel, bias):
    dr = jax.nn.softplus(decay_rate)

    pad = N_TPAD - N_IN
    t_pad = jnp.pad(times_in, (0, pad))
    s_pad = jnp.pad(segment_filter_ids, (0, pad), constant_values=-1)

    table = _build_table(dr, t_pad, s_pad).reshape(-1)
    ids_flat = (one_hot_predecessor_ids
                .reshape(_NB, 128, K, F_IN)
                .transpose(2, 0, 3, 1)
                .reshape(-1))
    g = _sc_gather(table, ids_flat)

    m3 = kernel
    a_mat = jnp.einsum("kcf,ts->sfktc", m3, jnp.eye(8, dtype=m3.dtype))
    a_mat = a_mat.reshape(8 * F_OUT, K * _QS)
    bias_col = jnp.tile(bias, (8,)).reshape(8 * F_OUT, 1)
    t_rep = jnp.repeat(times_out.reshape(_NB, 128), F_IN, axis=0)

    out4 = _finish(dr, g.reshape(K, F_IN * _NB, 128), t_rep, a_mat, bias_col)
    return jnp.transpose(out4, (1, 3, 0, 2)).reshape(N_OUT, F_OUT)


# speedup vs baseline: 1.3981x; 1.0334x over previous
import functools

import jax
import jax.numpy as jnp
from jax import lax
from jax.experimental import pallas as pl
from jax.experimental.pallas import tpu as pltpu
from jax.experimental.pallas import tpu_sc as plsc

N_IN = 262144
N_OUT = 262144
K = 5
F_IN = 4
F_OUT = 32
KC = K * F_IN

N_TPAD = N_IN + 1024
TOTAL = N_OUT * KC
NC, NS = 2, 16
NW = NC * NS
TOT_W = TOTAL // NW
CHUNK = 8192
GSUB = 128
N_CHUNKS = TOT_W // CHUNK


def _table_body(dr_ref, t_ref, s_ref, out_ref):
    t = t_ref[...]
    s = s_ref[...]
    arg = jnp.zeros_like(t)
    for c in range(F_IN):
        arg += jnp.where(s == c, dr_ref[c], 0.0)
    w = jnp.exp(arg * t)
    bits = lax.bitcast_convert_type(w, jnp.int32)
    packed = (bits & ~3) | (s & 3)
    out_ref[...] = jnp.where(s >= 0, packed, 0)


def _build_table(dr, t_pad, s_pad):
    rows = N_TPAD // 128
    return pl.pallas_call(
        _table_body,
        out_shape=jax.ShapeDtypeStruct((rows, 128), jnp.int32),
        in_specs=[
            pl.BlockSpec(memory_space=pltpu.MemorySpace.SMEM),
            pl.BlockSpec((rows, 128), lambda: (0, 0)),
            pl.BlockSpec((rows, 128), lambda: (0, 0)),
        ],
        out_specs=pl.BlockSpec((rows, 128), lambda: (0, 0)),
    )(dr, t_pad.reshape(rows, 128), s_pad.reshape(rows, 128))


def _gather_body(table_hbm, ids_hbm, out_hbm, idx0, idx1, rows0, rows1,
                 sem_i, sem_g0, sem_g1, sem_o0, sem_o1):
    wid = lax.axis_index("s") * NC + lax.axis_index("c")
    base = wid * TOT_W
    idx = (idx0, idx1)
    rows = (rows0, rows1)
    sem_g = (sem_g0, sem_g1)
    sem_o = (sem_o0, sem_o1)

    pltpu.async_copy(ids_hbm.at[pl.ds(base, CHUNK)], idx0, sem_i)

    def outer(oi, carry):
        for b in range(2):
            ci = oi * 2 + b
            off = base + ci * CHUNK
            pltpu.make_async_copy(ids_hbm.at[pl.ds(0, CHUNK)], idx[b],
                                  sem_i).wait()

            @pl.when(ci + 1 < N_CHUNKS)
            def _prefetch(off=off, nxt=idx[1 - b]):
                pltpu.async_copy(ids_hbm.at[pl.ds(off + CHUNK, CHUNK)], nxt,
                                 sem_i)

            @pl.when(ci >= 2)
            def _rows_free(rb=rows[b], so=sem_o[b]):
                pltpu.make_async_copy(ids_hbm.at[pl.ds(0, CHUNK)], rb,
                                      so).wait()

            def fire(m, c2, ib=idx[b], rb=rows[b], sg=sem_g[b]):
                pltpu.async_copy(
                    table_hbm.at[ib.at[pl.ds(m * GSUB, GSUB)]],
                    rb.at[pl.ds(m * GSUB, GSUB)],
                    sg,
                )
                return c2

            lax.fori_loop(0, CHUNK // GSUB, fire, 0)

            @pl.when(ci >= 1)
            def _finish_prev(pb=rows[1 - b], sg=sem_g[1 - b],
                             so=sem_o[1 - b], poff=off - CHUNK):
                pltpu.make_async_copy(table_hbm.at[pl.ds(0, CHUNK)], pb,
                                      sg).wait()
                pltpu.async_copy(pb, out_hbm.at[pl.ds(poff, CHUNK)], so)
        return carry

    lax.fori_loop(0, N_CHUNKS // 2, outer, 0)

    last = (N_CHUNKS - 1) % 2
    pltpu.make_async_copy(table_hbm.at[pl.ds(0, CHUNK)], rows[last],
                          sem_g[last]).wait()
    pltpu.async_copy(rows[last],
                     out_hbm.at[pl.ds(base + (N_CHUNKS - 1) * CHUNK, CHUNK)],
                     sem_o[last])
    pltpu.make_async_copy(ids_hbm.at[pl.ds(0, CHUNK)], rows0, sem_o0).wait()
    pltpu.make_async_copy(ids_hbm.at[pl.ds(0, CHUNK)], rows1, sem_o1).wait()


@functools.lru_cache(maxsize=None)
def _make_gather():
    return functools.partial(
        pl.kernel,
        mesh=plsc.VectorSubcoreMesh(
            core_axis_name="c", subcore_axis_name="s",
            num_cores=NC, num_subcores=NS,
        ),
        out_type=jax.ShapeDtypeStruct((TOTAL,), jnp.int32),
        scratch_types=[
            pltpu.VMEM((CHUNK,), jnp.int32),
            pltpu.VMEM((CHUNK,), jnp.int32),
            pltpu.VMEM((CHUNK,), jnp.int32),
            pltpu.VMEM((CHUNK,), jnp.int32),
            pltpu.SemaphoreType.DMA,
            pltpu.SemaphoreType.DMA,
            pltpu.SemaphoreType.DMA,
            pltpu.SemaphoreType.DMA,
            pltpu.SemaphoreType.DMA,
        ],
    )(_gather_body)


def _sc_gather(table, ids_flat):
    return _make_gather()(table, ids_flat)


_NB = N_OUT // 128
_SB = 128
_JBLK = _SB // F_IN
_QS = 32


def _out_body(dr_ref, g_ref, t_ref, a_ref, b_ref, o_ref):
    bits = g_ref[...]
    w = lax.bitcast_convert_type(bits & ~3, jnp.float32)
    ch = bits & 3
    cpat2 = lax.broadcasted_iota(jnp.int32, (_SB, 128), 0) % F_IN
    t = t_ref[...]
    drsel = jnp.zeros((_SB, 128), jnp.float32)
    for c in range(F_IN):
        drsel += jnp.where(cpat2 == c, dr_ref[c], 0.0)
    e = jnp.exp(-drsel * t)
    vals = jnp.where(ch == cpat2[None], w * e[None], 0.0)
    a = a_ref[...]
    b = b_ref[...]
    for q in range(_SB // _QS):
        vq = vals[:, q * _QS:(q + 1) * _QS, :]
        v160 = vq.reshape(K * _QS, 128)
        res = lax.dot_general(a, v160, (((1,), (0,)), ((), ())),
                              preferred_element_type=jnp.float32)
        res = res + b
        res4 = res.reshape(8, F_OUT // 8, 8, 128)
        o_ref[:, q * 8:(q + 1) * 8, :, :] = jnp.transpose(res4, (1, 0, 2, 3))


def _finish(dr, g3, t_rep, a_mat, bias_col):
    grid = (F_IN * _NB // _SB,)
    return pl.pallas_call(
        _out_body,
        grid=grid,
        out_shape=jax.ShapeDtypeStruct((F_OUT // 8, _NB, 8, 128), jnp.float32),
        in_specs=[
            pl.BlockSpec(memory_space=pltpu.MemorySpace.SMEM),
            pl.BlockSpec((K, _SB, 128), lambda i: (0, i, 0)),
            pl.BlockSpec((_SB, 128), lambda i: (i, 0)),
            pl.BlockSpec((8 * F_OUT, K * _QS), lambda i: (0, 0)),
            pl.BlockSpec((8 * F_OUT, 1), lambda i: (0, 0)),
        ],
        out_specs=pl.BlockSpec((F_OUT // 8, _JBLK, 8, 128), lambda i: (0, i, 0, 0)),
    )(dr, g3, t_rep, a_mat, bias_col)


def kernel(times_in, times_out, segment_filter_ids, one_hot_predecessor_ids,
           decay_rate, kernel, bias):
    dr = jax.nn.softplus(decay_rate)

    pad = N_TPAD - N_IN
    t_pad = jnp.pad(times_in, (0, pad))
    s_pad = jnp.pad(segment_filter_ids, (0, pad), constant_values=-1)

    table = _build_table(dr, t_pad, s_pad).reshape(-1)
    ids_flat = (one_hot_predecessor_ids
                .reshape(_NB, 128, K, F_IN)
                .transpose(2, 0, 3, 1)
                .reshape(-1))
    g = _sc_gather(table, ids_flat)

    m3 = kernel
    a_mat = jnp.einsum("kcf,ts->sfktc", m3, jnp.eye(8, dtype=m3.dtype))
    a_mat = a_mat.reshape(8 * F_OUT, K * _QS)
    bias_col = jnp.tile(bias, (8,)).reshape(8 * F_OUT, 1)
    t_rep = jnp.repeat(times_out.reshape(_NB, 128), F_IN, axis=0)

    out4 = _finish(dr, g.reshape(K, F_IN * _NB, 128), t_rep, a_mat, bias_col)
    return jnp.transpose(out4, (1, 3, 0, 2)).reshape(N_OUT, F_OUT)


# speedup vs baseline: 1.4046x; 1.0047x over previous
import functools

import jax
import jax.numpy as jnp
from jax import lax
from jax.experimental import pallas as pl
from jax.experimental.pallas import tpu as pltpu
from jax.experimental.pallas import tpu_sc as plsc

N_IN = 262144
N_OUT = 262144
K = 5
F_IN = 4
F_OUT = 32
KC = K * F_IN

N_TPAD = N_IN + 1024
TOTAL = N_OUT * KC
NC, NS = 2, 16
NW = NC * NS
TOT_W = TOTAL // NW
CHUNK = 8192
GSUB = 128
N_CHUNKS = TOT_W // CHUNK


def _table_body(dr_ref, t_ref, s_ref, out_ref):
    t = t_ref[...]
    s = s_ref[...]
    arg = jnp.zeros_like(t)
    for c in range(F_IN):
        arg += jnp.where(s == c, dr_ref[c], 0.0)
    w = jnp.exp(arg * t)
    bits = lax.bitcast_convert_type(w, jnp.int32)
    packed = (bits & ~3) | (s & 3)
    out_ref[...] = jnp.where(s >= 0, packed, 0)


def _build_table(dr, t_pad, s_pad):
    rows = N_TPAD // 128
    return pl.pallas_call(
        _table_body,
        out_shape=jax.ShapeDtypeStruct((rows, 128), jnp.int32),
        in_specs=[
            pl.BlockSpec(memory_space=pltpu.MemorySpace.SMEM),
            pl.BlockSpec((rows, 128), lambda: (0, 0)),
            pl.BlockSpec((rows, 128), lambda: (0, 0)),
        ],
        out_specs=pl.BlockSpec((rows, 128), lambda: (0, 0)),
    )(dr, t_pad.reshape(rows, 128), s_pad.reshape(rows, 128))


def _gather_body(table_hbm, ids_hbm, out_hbm, idx0, idx1, rows0, rows1,
                 sem_i, sem_g0, sem_g1, sem_o0, sem_o1):
    wid = lax.axis_index("s") * NC + lax.axis_index("c")
    base = wid * TOT_W
    idx = (idx0, idx1)
    rows = (rows0, rows1)
    sem_g = (sem_g0, sem_g1)
    sem_o = (sem_o0, sem_o1)

    pltpu.async_copy(ids_hbm.at[pl.ds(base, CHUNK)], idx0, sem_i)

    def outer(oi, carry):
        for b in range(2):
            ci = oi * 2 + b
            off = base + ci * CHUNK
            pltpu.make_async_copy(ids_hbm.at[pl.ds(0, CHUNK)], idx[b],
                                  sem_i).wait()

            @pl.when(ci >= 2)
            def _rows_free(rb=rows[b], so=sem_o[b]):
                pltpu.make_async_copy(ids_hbm.at[pl.ds(0, CHUNK)], rb,
                                      so).wait()

            def fire(m, c2, ib=idx[b], rb=rows[b], sg=sem_g[b]):
                pltpu.async_copy(
                    table_hbm.at[ib.at[pl.ds(m * GSUB, GSUB)]],
                    rb.at[pl.ds(m * GSUB, GSUB)],
                    sg,
                )
                return c2

            lax.fori_loop(0, CHUNK // GSUB, fire, 0)

            @pl.when(ci >= 1)
            def _finish_prev(pb=rows[1 - b], sg=sem_g[1 - b],
                             so=sem_o[1 - b], poff=off - CHUNK):
                pltpu.make_async_copy(table_hbm.at[pl.ds(0, CHUNK)], pb,
                                      sg).wait()
                pltpu.async_copy(pb, out_hbm.at[pl.ds(poff, CHUNK)], so)

            @pl.when(ci + 1 < N_CHUNKS)
            def _prefetch(off=off, nxt=idx[1 - b]):
                pltpu.async_copy(ids_hbm.at[pl.ds(off + CHUNK, CHUNK)], nxt,
                                 sem_i)
        return carry

    lax.fori_loop(0, N_CHUNKS // 2, outer, 0)

    last = (N_CHUNKS - 1) % 2
    pltpu.make_async_copy(table_hbm.at[pl.ds(0, CHUNK)], rows[last],
                          sem_g[last]).wait()
    pltpu.async_copy(rows[last],
                     out_hbm.at[pl.ds(base + (N_CHUNKS - 1) * CHUNK, CHUNK)],
                     sem_o[last])
    pltpu.make_async_copy(ids_hbm.at[pl.ds(0, CHUNK)], rows0, sem_o0).wait()
    pltpu.make_async_copy(ids_hbm.at[pl.ds(0, CHUNK)], rows1, sem_o1).wait()


@functools.lru_cache(maxsize=None)
def _make_gather():
    return functools.partial(
        pl.kernel,
        mesh=plsc.VectorSubcoreMesh(
            core_axis_name="c", subcore_axis_name="s",
            num_cores=NC, num_subcores=NS,
        ),
        out_type=jax.ShapeDtypeStruct((TOTAL,), jnp.int32),
        scratch_types=[
            pltpu.VMEM((CHUNK,), jnp.int32),
            pltpu.VMEM((CHUNK,), jnp.int32),
            pltpu.VMEM((CHUNK,), jnp.int32),
            pltpu.VMEM((CHUNK,), jnp.int32),
            pltpu.SemaphoreType.DMA,
            pltpu.SemaphoreType.DMA,
            pltpu.SemaphoreType.DMA,
            pltpu.SemaphoreType.DMA,
            pltpu.SemaphoreType.DMA,
        ],
    )(_gather_body)


def _sc_gather(table, ids_flat):
    return _make_gather()(table, ids_flat)


_NB = N_OUT // 128
_SB = 128
_JBLK = _SB // F_IN
_QS = 32


def _out_body(dr_ref, g_ref, t_ref, a_ref, b_ref, o_ref):
    bits = g_ref[...]
    w = lax.bitcast_convert_type(bits & ~3, jnp.float32)
    ch = bits & 3
    cpat2 = lax.broadcasted_iota(jnp.int32, (_SB, 128), 0) % F_IN
    t = t_ref[...]
    drsel = jnp.zeros((_SB, 128), jnp.float32)
    for c in range(F_IN):
        drsel += jnp.where(cpat2 == c, dr_ref[c], 0.0)
    e = jnp.exp(-drsel * t)
    vals = jnp.where(ch == cpat2[None], w * e[None], 0.0)
    a = a_ref[...]
    b = b_ref[...]
    for q in range(_SB // _QS):
        vq = vals[:, q * _QS:(q + 1) * _QS, :]
        v160 = vq.reshape(K * _QS, 128)
        res = lax.dot_general(a, v160, (((1,), (0,)), ((), ())),
                              preferred_element_type=jnp.float32)
        res = res + b
        res4 = res.reshape(8, F_OUT // 8, 8, 128)
        o_ref[:, q * 8:(q + 1) * 8, :, :] = jnp.transpose(res4, (1, 0, 2, 3))


def _finish(dr, g3, t_rep, a_mat, bias_col):
    grid = (F_IN * _NB // _SB,)
    return pl.pallas_call(
        _out_body,
        grid=grid,
        out_shape=jax.ShapeDtypeStruct((F_OUT // 8, _NB, 8, 128), jnp.float32),
        in_specs=[
            pl.BlockSpec(memory_space=pltpu.MemorySpace.SMEM),
            pl.BlockSpec((K, _SB, 128), lambda i: (0, i, 0)),
            pl.BlockSpec((_SB, 128), lambda i: (i, 0)),
            pl.BlockSpec((8 * F_OUT, K * _QS), lambda i: (0, 0)),
            pl.BlockSpec((8 * F_OUT, 1), lambda i: (0, 0)),
        ],
        out_specs=pl.BlockSpec((F_OUT // 8, _JBLK, 8, 128), lambda i: (0, i, 0, 0)),
    )(dr, g3, t_rep, a_mat, bias_col)


def kernel(times_in, times_out, segment_filter_ids, one_hot_predecessor_ids,
           decay_rate, kernel, bias):
    dr = jax.nn.softplus(decay_rate)

    pad = N_TPAD - N_IN
    t_pad = jnp.pad(times_in, (0, pad))
    s_pad = jnp.pad(segment_filter_ids, (0, pad), constant_values=-1)

    table = _build_table(dr, t_pad, s_pad).reshape(-1)
    ids_flat = (one_hot_predecessor_ids
                .reshape(_NB, 128, K, F_IN)
                .transpose(2, 0, 3, 1)
                .reshape(-1))
    g = _sc_gather(table, ids_flat)

    m3 = kernel
    a_mat = jnp.einsum("kcf,ts->sfktc", m3, jnp.eye(8, dtype=m3.dtype))
    a_mat = a_mat.reshape(8 * F_OUT, K * _QS)
    bias_col = jnp.tile(bias, (8,)).reshape(8 * F_OUT, 1)
    t_rep = jnp.repeat(times_out.reshape(_NB, 128), F_IN, axis=0)

    out4 = _finish(dr, g.reshape(K, F_IN * _NB, 128), t_rep, a_mat, bias_col)
    return jnp.transpose(out4, (1, 3, 0, 2)).reshape(N_OUT, F_OUT)


# speedup vs baseline: 1.4061x; 1.0011x over previous
import functools

import jax
import jax.numpy as jnp
from jax import lax
from jax.experimental import pallas as pl
from jax.experimental.pallas import tpu as pltpu
from jax.experimental.pallas import tpu_sc as plsc

N_IN = 262144
N_OUT = 262144
K = 5
F_IN = 4
F_OUT = 32
KC = K * F_IN

N_TPAD = N_IN + 1024
TOTAL = N_OUT * KC
NC, NS = 2, 16
NW = NC * NS
TOT_W = TOTAL // NW
CHUNK = 16384
GSUB = 256
N_CHUNKS = TOT_W // CHUNK


def _table_body(dr_ref, t_ref, s_ref, out_ref):
    t = t_ref[...]
    s = s_ref[...]
    arg = jnp.zeros_like(t)
    for c in range(F_IN):
        arg += jnp.where(s == c, dr_ref[c], 0.0)
    w = jnp.exp(arg * t)
    bits = lax.bitcast_convert_type(w, jnp.int32)
    packed = (bits & ~3) | (s & 3)
    out_ref[...] = jnp.where(s >= 0, packed, 0)


def _build_table(dr, t_pad, s_pad):
    rows = N_TPAD // 128
    return pl.pallas_call(
        _table_body,
        out_shape=jax.ShapeDtypeStruct((rows, 128), jnp.int32),
        in_specs=[
            pl.BlockSpec(memory_space=pltpu.MemorySpace.SMEM),
            pl.BlockSpec((rows, 128), lambda: (0, 0)),
            pl.BlockSpec((rows, 128), lambda: (0, 0)),
        ],
        out_specs=pl.BlockSpec((rows, 128), lambda: (0, 0)),
    )(dr, t_pad.reshape(rows, 128), s_pad.reshape(rows, 128))


def _gather_body(table_hbm, ids_hbm, out_hbm, idx0, idx1, rows0, rows1,
                 sem_i, sem_g0, sem_g1, sem_o0, sem_o1):
    wid = lax.axis_index("s") * NC + lax.axis_index("c")
    base = wid * TOT_W
    idx = (idx0, idx1)
    rows = (rows0, rows1)
    sem_g = (sem_g0, sem_g1)
    sem_o = (sem_o0, sem_o1)

    pltpu.async_copy(ids_hbm.at[pl.ds(base, CHUNK)], idx0, sem_i)

    def outer(oi, carry):
        for b in range(2):
            ci = oi * 2 + b
            off = base + ci * CHUNK
            pltpu.make_async_copy(ids_hbm.at[pl.ds(0, CHUNK)], idx[b],
                                  sem_i).wait()

            @pl.when(ci >= 2)
            def _rows_free(rb=rows[b], so=sem_o[b]):
                pltpu.make_async_copy(ids_hbm.at[pl.ds(0, CHUNK)], rb,
                                      so).wait()

            def fire(m, c2, ib=idx[b], rb=rows[b], sg=sem_g[b]):
                pltpu.async_copy(
                    table_hbm.at[ib.at[pl.ds(m * GSUB, GSUB)]],
                    rb.at[pl.ds(m * GSUB, GSUB)],
                    sg,
                )
                return c2

            lax.fori_loop(0, CHUNK // GSUB, fire, 0)

            @pl.when(ci >= 1)
            def _finish_prev(pb=rows[1 - b], sg=sem_g[1 - b],
                             so=sem_o[1 - b], poff=off - CHUNK):
                pltpu.make_async_copy(table_hbm.at[pl.ds(0, CHUNK)], pb,
                                      sg).wait()
                pltpu.async_copy(pb, out_hbm.at[pl.ds(poff, CHUNK)], so)

            @pl.when(ci + 1 < N_CHUNKS)
            def _prefetch(off=off, nxt=idx[1 - b]):
                pltpu.async_copy(ids_hbm.at[pl.ds(off + CHUNK, CHUNK)], nxt,
                                 sem_i)
        return carry

    lax.fori_loop(0, N_CHUNKS // 2, outer, 0)

    last = (N_CHUNKS - 1) % 2
    pltpu.make_async_copy(table_hbm.at[pl.ds(0, CHUNK)], rows[last],
                          sem_g[last]).wait()
    pltpu.async_copy(rows[last],
                     out_hbm.at[pl.ds(base + (N_CHUNKS - 1) * CHUNK, CHUNK)],
                     sem_o[last])
    pltpu.make_async_copy(ids_hbm.at[pl.ds(0, CHUNK)], rows0, sem_o0).wait()
    pltpu.make_async_copy(ids_hbm.at[pl.ds(0, CHUNK)], rows1, sem_o1).wait()


@functools.lru_cache(maxsize=None)
def _make_gather():
    return functools.partial(
        pl.kernel,
        mesh=plsc.VectorSubcoreMesh(
            core_axis_name="c", subcore_axis_name="s",
            num_cores=NC, num_subcores=NS,
        ),
        out_type=jax.ShapeDtypeStruct((TOTAL,), jnp.int32),
        scratch_types=[
            pltpu.VMEM((CHUNK,), jnp.int32),
            pltpu.VMEM((CHUNK,), jnp.int32),
            pltpu.VMEM((CHUNK,), jnp.int32),
            pltpu.VMEM((CHUNK,), jnp.int32),
            pltpu.SemaphoreType.DMA,
            pltpu.SemaphoreType.DMA,
            pltpu.SemaphoreType.DMA,
            pltpu.SemaphoreType.DMA,
            pltpu.SemaphoreType.DMA,
        ],
    )(_gather_body)


def _sc_gather(table, ids_flat):
    return _make_gather()(table, ids_flat)


_NB = N_OUT // 128
_SB = 128
_JBLK = _SB // F_IN
_QS = 32


def _out_body(dr_ref, g_ref, t_ref, a_ref, b_ref, o_ref):
    bits = g_ref[...]
    w = lax.bitcast_convert_type(bits & ~3, jnp.float32)
    ch = bits & 3
    cpat2 = lax.broadcasted_iota(jnp.int32, (_SB, 128), 0) % F_IN
    t = t_ref[...]
    drsel = jnp.zeros((_SB, 128), jnp.float32)
    for c in range(F_IN):
        drsel += jnp.where(cpat2 == c, dr_ref[c], 0.0)
    e = jnp.exp(-drsel * t)
    vals = jnp.where(ch == cpat2[None], w * e[None], 0.0)
    a = a_ref[...]
    b = b_ref[...]
    for q in range(_SB // _QS):
        vq = vals[:, q * _QS:(q + 1) * _QS, :]
        v160 = vq.reshape(K * _QS, 128)
        res = lax.dot_general(a, v160, (((1,), (0,)), ((), ())),
                              preferred_element_type=jnp.float32)
        res = res + b
        res4 = res.reshape(8, F_OUT // 8, 8, 128)
        o_ref[:, q * 8:(q + 1) * 8, :, :] = jnp.transpose(res4, (1, 0, 2, 3))


def _finish(dr, g3, t_rep, a_mat, bias_col):
    grid = (F_IN * _NB // _SB,)
    return pl.pallas_call(
        _out_body,
        grid=grid,
        out_shape=jax.ShapeDtypeStruct((F_OUT // 8, _NB, 8, 128), jnp.float32),
        in_specs=[
            pl.BlockSpec(memory_space=pltpu.MemorySpace.SMEM),
            pl.BlockSpec((K, _SB, 128), lambda i: (0, i, 0)),
            pl.BlockSpec((_SB, 128), lambda i: (i, 0)),
            pl.BlockSpec((8 * F_OUT, K * _QS), lambda i: (0, 0)),
            pl.BlockSpec((8 * F_OUT, 1), lambda i: (0, 0)),
        ],
        out_specs=pl.BlockSpec((F_OUT // 8, _JBLK, 8, 128), lambda i: (0, i, 0, 0)),
    )(dr, g3, t_rep, a_mat, bias_col)


def kernel(times_in, times_out, segment_filter_ids, one_hot_predecessor_ids,
           decay_rate, kernel, bias):
    dr = jax.nn.softplus(decay_rate)

    pad = N_TPAD - N_IN
    t_pad = jnp.pad(times_in, (0, pad))
    s_pad = jnp.pad(segment_filter_ids, (0, pad), constant_values=-1)

    table = _build_table(dr, t_pad, s_pad).reshape(-1)
    ids_flat = (one_hot_predecessor_ids
                .reshape(_NB, 128, K, F_IN)
                .transpose(2, 0, 3, 1)
                .reshape(-1))
    g = _sc_gather(table, ids_flat)

    m3 = kernel
    a_mat = jnp.einsum("kcf,ts->sfktc", m3, jnp.eye(8, dtype=m3.dtype))
    a_mat = a_mat.reshape(8 * F_OUT, K * _QS)
    bias_col = jnp.tile(bias, (8,)).reshape(8 * F_OUT, 1)
    t_rep = jnp.repeat(times_out.reshape(_NB, 128), F_IN, axis=0)

    out4 = _finish(dr, g.reshape(K, F_IN * _NB, 128), t_rep, a_mat, bias_col)
    return jnp.transpose(out4, (1, 3, 0, 2)).reshape(N_OUT, F_OUT)
